# baseline (speedup 1.0000x reference)
import functools
import jax
import jax.numpy as jnp
from jax import lax
from jax.experimental import pallas as pl
from jax.experimental.pallas import tpu as pltpu

NU = 10000
NI = 10000
E = 160000
DIN = 384
H = 256
BM = 1000


def _proj_body(x_ref, w_ref, b_ref, o_ref):
    o_ref[:] = jnp.dot(x_ref[:], w_ref[:], preferred_element_type=jnp.float32) + b_ref[:]


def _proj(x, w, b):
    n, k = x.shape
    h = w.shape[1]
    grid = (n // BM,)
    return pl.pallas_call(
        _proj_body,
        grid=grid,
        in_specs=[
            pl.BlockSpec((BM, k), lambda m: (m, 0)),
            pl.BlockSpec((k, h), lambda m: (0, 0)),
            pl.BlockSpec((1, h), lambda m: (0, 0)),
        ],
        out_specs=pl.BlockSpec((BM, h), lambda m: (m, 0)),
        out_shape=jax.ShapeDtypeStruct((n, h), jnp.float32),
    )(x, w, b.reshape(1, h))


def _left_body(h_ref, w_ref, o_ref):
    o_ref[:] = jnp.dot(h_ref[:], w_ref[:], preferred_element_type=jnp.float32)


def _left_proj(h, wl):
    n = h.shape[0]
    nm = n // BM
    return pl.pallas_call(
        _left_body,
        grid=(2, nm),
        in_specs=[
            pl.BlockSpec((BM, H), lambda c, m: (m, 0)),
            pl.BlockSpec((H, 128), lambda c, m: (0, c)),
        ],
        out_specs=pl.BlockSpec((BM, 128), lambda c, m: (c * nm + m, 0)),
        out_shape=jax.ShapeDtypeStruct((2 * n, 128), jnp.float32),
    )(h, wl)


def _epi_body(s_ref, inv_ref, h_ref, w_ref, b_ref, o_ref):
    right = jnp.dot(h_ref[:], w_ref[:], preferred_element_type=jnp.float32) + b_ref[:]
    left = jnp.concatenate([s_ref[0], s_ref[1]], axis=1) * inv_ref[:]
    o_ref[:] = jnp.maximum(left + right, 0.0)


def _epilogue(s2, inv, h, wr, b):
    n = h.shape[0]
    return pl.pallas_call(
        _epi_body,
        grid=(n // BM,),
        in_specs=[
            pl.BlockSpec((2, BM, 128), lambda m: (0, m, 0)),
            pl.BlockSpec((BM, 1), lambda m: (m, 0)),
            pl.BlockSpec((BM, H), lambda m: (m, 0)),
            pl.BlockSpec((H, H), lambda m: (0, 0)),
            pl.BlockSpec((1, H), lambda m: (0, 0)),
        ],
        out_specs=pl.BlockSpec((BM, H), lambda m: (m, 0)),
        out_shape=jax.ShapeDtypeStruct((n, H), jnp.float32),
    )(s2, inv, h, wr, b.reshape(1, H))


def _segsum(p2, src, dst, num):
    gathered0 = p2[src]
    gathered1 = p2[src + num]
    s0 = jax.ops.segment_sum(gathered0, dst, num_segments=num)
    s1 = jax.ops.segment_sum(gathered1, dst, num_segments=num)
    return jnp.stack([s0, s1])


def kernel(x_user, x_item, ei_u2i, ei_i2u, Win_u, bin_u, Win_i, bin_i, Wl0_u2i, bl0_u2i, Wr0_u2i, br0_u2i, Wl0_i2u, bl0_i2u, Wr0_i2u, br0_i2u, Wl1_u2i, bl1_u2i, Wr1_u2i, br1_u2i, Wl1_i2u, bl1_i2u, Wr1_i2u, br1_i2u):
    src_u2i, dst_u2i = ei_u2i[0], ei_u2i[1]
    src_i2u, dst_i2u = ei_i2u[0], ei_i2u[1]

    cnt_i = jax.ops.segment_sum(jnp.ones((E,), jnp.float32), dst_u2i, num_segments=NI)
    cnt_u = jax.ops.segment_sum(jnp.ones((E,), jnp.float32), dst_i2u, num_segments=NU)
    inv_i = (1.0 / jnp.maximum(cnt_i, 1.0)).reshape(NI, 1)
    inv_u = (1.0 / jnp.maximum(cnt_u, 1.0)).reshape(NU, 1)

    hu = _proj(x_user, Win_u, bin_u)
    hi = _proj(x_item, Win_i, bin_i)

    layers = [
        (Wl0_u2i, bl0_u2i, Wr0_u2i, br0_u2i, Wl0_i2u, bl0_i2u, Wr0_i2u, br0_i2u),
        (Wl1_u2i, bl1_u2i, Wr1_u2i, br1_u2i, Wl1_i2u, bl1_i2u, Wr1_i2u, br1_i2u),
    ]
    for (Wlu2i, blu2i, Wru2i, bru2i, Wli2u, bli2u, Wri2u, bri2u) in layers:
        pi = _left_proj(hu, Wlu2i)
        si = _segsum(pi, src_u2i, dst_u2i, NI)
        pu = _left_proj(hi, Wli2u)
        su = _segsum(pu, src_i2u, dst_i2u, NU)
        new_i = _epilogue(si, inv_i, hi, Wru2i, blu2i + bru2i)
        new_u = _epilogue(su, inv_u, hu, Wri2u, bli2u + bri2u)
        hu, hi = new_u, new_i
    return hu, hi


# speedup vs baseline: 5.1361x; 5.1361x over previous
import functools
import jax
import jax.numpy as jnp
from jax import lax
from jax.experimental import pallas as pl
from jax.experimental.pallas import tpu as pltpu
from jax.experimental.pallas import tpu_sc as plsc

NU = 10000
NI = 10000
E = 160000
DIN = 384
H = 256
BM = 1000

NS = 16
NC = 2
EC = 125
NCH = E // (NS * EC)
KB = 16
ROWS_PER_TILE = NU // NS


def _proj_body(x_ref, w_ref, b_ref, o_ref):
    o_ref[:] = jnp.dot(x_ref[:], w_ref[:], preferred_element_type=jnp.float32) + b_ref[:]


def _proj(x, w, b):
    n, k = x.shape
    h = w.shape[1]
    return pl.pallas_call(
        _proj_body,
        grid=(n // BM,),
        in_specs=[
            pl.BlockSpec((BM, k), lambda m: (m, 0)),
            pl.BlockSpec((k, h), lambda m: (0, 0)),
            pl.BlockSpec((1, h), lambda m: (0, 0)),
        ],
        out_specs=pl.BlockSpec((BM, h), lambda m: (m, 0)),
        out_shape=jax.ShapeDtypeStruct((n, h), jnp.float32),
    )(x, w, b.reshape(1, h))


def _left_body(h_ref, w_ref, o_ref):
    o_ref[:] = jnp.dot(h_ref[:], w_ref[:], preferred_element_type=jnp.float32)


def _left_proj(h, wl):
    n = h.shape[0]
    nm = n // BM
    return pl.pallas_call(
        _left_body,
        grid=(2, nm),
        in_specs=[
            pl.BlockSpec((BM, H), lambda c, m: (m, 0)),
            pl.BlockSpec((H, 128), lambda c, m: (0, c)),
        ],
        out_specs=pl.BlockSpec((BM, 128), lambda c, m: (c * nm + m, 0)),
        out_shape=jax.ShapeDtypeStruct((2 * n, 128), jnp.float32),
    )(h, wl)


def _epi_body(s_ref, inv_ref, h_ref, w_ref, b_ref, o_ref):
    right = jnp.dot(h_ref[:], w_ref[:], preferred_element_type=jnp.float32) + b_ref[:]
    left = jnp.concatenate([s_ref[0], s_ref[1]], axis=1) * inv_ref[:]
    o_ref[:] = jnp.maximum(left + right, 0.0)


def _epilogue(s2, inv, h, wr, b):
    n = h.shape[0]
    return pl.pallas_call(
        _epi_body,
        grid=(n // BM,),
        in_specs=[
            pl.BlockSpec((2, BM, 128), lambda m: (0, m, 0)),
            pl.BlockSpec((BM, 1), lambda m: (m, 0)),
            pl.BlockSpec((BM, H), lambda m: (m, 0)),
            pl.BlockSpec((H, H), lambda m: (0, 0)),
            pl.BlockSpec((1, H), lambda m: (0, 0)),
        ],
        out_specs=pl.BlockSpec((BM, H), lambda m: (m, 0)),
        out_shape=jax.ShapeDtypeStruct((n, H), jnp.float32),
    )(s2, inv, h, wr, b.reshape(1, H))


_MESH = plsc.VectorSubcoreMesh(core_axis_name="c", subcore_axis_name="s")


def _segsum_body(p2, srcs, dsts, zeros, out, acc, srcv, dstv, rows, gsem, gsem2):
    c = lax.axis_index("c")
    s = lax.axis_index("s")
    @pl.when(s < 10)
    def _():
        pltpu.sync_copy(zeros, acc.at[pl.ds(s * 1000, 1000)])
    plsc.subcore_barrier()

    def block(kb, carry):
        pltpu.sync_copy(srcs.at[c, s, pl.ds(kb * KB, KB)], srcv)
        pltpu.sync_copy(dsts.at[s, pl.ds(kb * KB, KB)], dstv)

        def step(t, carry2):
            g0 = pltpu.async_copy(p2.at[srcv.at[2 * t]], rows.at[0], gsem)
            g1 = pltpu.async_copy(p2.at[srcv.at[2 * t + 1]], rows.at[1], gsem2)
            g0.wait()
            pltpu.sync_copy(rows.at[0], acc.at[dstv.at[2 * t]], add=True)
            g1.wait()
            pltpu.sync_copy(rows.at[1], acc.at[dstv.at[2 * t + 1]], add=True)
            return carry2

        lax.fori_loop(0, KB // 2, step, 0)
        return carry

    lax.fori_loop(0, NCH // KB, block, 0)
    plsc.subcore_barrier()
    @pl.when(s < 10)
    def _():
        pltpu.sync_copy(acc.at[pl.ds(s * 1000, 1000)],
                        out.at[pl.ds(c * NU + s * 1000, 1000)])


_segsum_call = pl.kernel(
    _segsum_body,
    out_type=jax.ShapeDtypeStruct((2 * NU, 128), jnp.float32),
    mesh=_MESH,
    scratch_types=[
        pltpu.VMEM_SHARED((NU, 128), jnp.float32),
        pltpu.VMEM((KB, EC), jnp.int32),
        pltpu.VMEM((KB, EC), jnp.int32),
        pltpu.VMEM((2, EC, 128), jnp.float32),
        pltpu.SemaphoreType.DMA,
        pltpu.SemaphoreType.DMA,
    ],
)


def _segsum(p2, srcs3, dsts3, zeros):
    return _segsum_call(p2, srcs3, dsts3, zeros).reshape(2, NU, 128)


_EPT = E // NS


def _counts_body(dsts, ones_hbm, zeros, out, acc, dstv, ones_buf):
    c = lax.axis_index("c")
    s = lax.axis_index("s")

    @pl.when(s < 10)
    def _():
        pltpu.sync_copy(zeros, acc.at[pl.ds(s * 1000, 1000)])
    pltpu.sync_copy(ones_hbm, ones_buf)
    pltpu.sync_copy(dsts.at[c, s], dstv)
    plsc.subcore_barrier()

    def step(t, carry):
        pltpu.sync_copy(ones_buf, acc.at[dstv.at[t]], add=True)
        return carry

    lax.fori_loop(0, NCH, step, 0)
    plsc.subcore_barrier()
    @pl.when(s < 10)
    def _():
        pltpu.sync_copy(acc.at[pl.ds(s * 1000, 1000)], out.at[c, pl.ds(s * 1000, 1000)])


_counts_call = pl.kernel(
    _counts_body,
    out_type=jax.ShapeDtypeStruct((2, NU, 128), jnp.float32),
    mesh=_MESH,
    scratch_types=[
        pltpu.VMEM_SHARED((NU, 128), jnp.float32),
        pltpu.VMEM((NCH, EC), jnp.int32),
        pltpu.VMEM((EC, 128), jnp.float32),
    ],
)


def kernel(x_user, x_item, ei_u2i, ei_i2u, Win_u, bin_u, Win_i, bin_i, Wl0_u2i, bl0_u2i, Wr0_u2i, br0_u2i, Wl0_i2u, bl0_i2u, Wr0_i2u, br0_i2u, Wl1_u2i, bl1_u2i, Wr1_u2i, br1_u2i, Wl1_i2u, bl1_i2u, Wr1_i2u, br1_i2u):
    def prep(ei, n_src):
        src = ei[0].reshape(NS, NCH, EC)
        srcs3 = jnp.stack([src, src + n_src])
        dsts3 = ei[1].reshape(NS, NCH, EC)
        return srcs3, dsts3

    srcs_u2i, dsts_u2i = prep(ei_u2i, NU)
    srcs_i2u, dsts_i2u = prep(ei_i2u, NI)
    zeros = jnp.zeros((1000, 128), jnp.float32)

    dst_both = jnp.stack([dsts_u2i, dsts_i2u])
    ones_p = jnp.ones((EC, 128), jnp.float32)
    cnts = _counts_call(dst_both, ones_p, zeros)
    inv_i = (1.0 / jnp.maximum(cnts[0, :, 0], 1.0)).reshape(NI, 1)
    inv_u = (1.0 / jnp.maximum(cnts[1, :, 0], 1.0)).reshape(NU, 1)

    hu = _proj(x_user, Win_u, bin_u)
    hi = _proj(x_item, Win_i, bin_i)

    layers = [
        (Wl0_u2i, bl0_u2i, Wr0_u2i, br0_u2i, Wl0_i2u, bl0_i2u, Wr0_i2u, br0_i2u),
        (Wl1_u2i, bl1_u2i, Wr1_u2i, br1_u2i, Wl1_i2u, bl1_i2u, Wr1_i2u, br1_i2u),
    ]
    for (Wlu2i, blu2i, Wru2i, bru2i, Wli2u, bli2u, Wri2u, bri2u) in layers:
        pi = _left_proj(hu, Wlu2i)
        si = _segsum(pi, srcs_u2i, dsts_u2i, zeros)
        pu = _left_proj(hi, Wli2u)
        su = _segsum(pu, srcs_i2u, dsts_i2u, zeros)
        new_i = _epilogue(si, inv_i, hi, Wru2i, blu2i + bru2i)
        new_u = _epilogue(su, inv_u, hu, Wri2u, bli2u + bri2u)
        hu, hi = new_u, new_i
    return hu, hi
